# baseline (speedup 1.0000x reference)
import functools

import jax
import jax.numpy as jnp
from jax import lax
from jax.experimental import pallas as pl
from jax.experimental.pallas import tpu as pltpu
from jax.experimental.pallas import tpu_sc as plsc

VOCAB = 1000000
SEQ_LEN = 200
D_MODEL = 64
BATCH = 4096


def _sc_call(idxT, tab2, pos_enc):
    info = plsc.get_sparse_core_info()
    nc, ns = info.num_cores, info.num_subcores
    nw = nc * ns
    bcol = BATCH // nw

    mesh = plsc.VectorSubcoreMesh(core_axis_name="c", subcore_axis_name="s")

    @functools.partial(
        pl.kernel,
        out_type=jax.ShapeDtypeStruct((SEQ_LEN, D_MODEL, BATCH), jnp.float32),
        mesh=mesh,
        scratch_types=[
            pltpu.VMEM((bcol,), jnp.int32),
            pltpu.VMEM((bcol,), jnp.int32),
            pltpu.VMEM((bcol,), jnp.int32),
            pltpu.VMEM((bcol, 128), jnp.float32),
            pltpu.VMEM((D_MODEL, bcol), jnp.float32),
            pltpu.VMEM((SEQ_LEN, D_MODEL), jnp.float32),
            pltpu.SemaphoreType.DMA,
        ],
        compiler_params=pltpu.CompilerParams(
            use_tc_tiling_on_sc=True, needs_layout_passes=False),
    )
    def k(idxT_hbm, tab2_hbm, pos_hbm, out_hbm, idx_v, grp_v, off_v,
          gath_v, outt_v, pos_v, sem):
        wid = lax.axis_index("s") * nc + lax.axis_index("c")
        b0 = pl.multiple_of(wid * bcol, bcol)
        pltpu.sync_copy(pos_hbm, pos_v)

        def sloop(s, carry):
            pltpu.sync_copy(idxT_hbm.at[s, pl.ds(b0, bcol)], idx_v)

            def shf(i, c2):
                v = idx_v[pl.ds(i * 16, 16)]
                grp_v[pl.ds(i * 16, 16)] = lax.shift_right_logical(v, 1)
                off_v[pl.ds(i * 16, 16)] = lax.shift_left(
                    lax.bitwise_and(v, 1), 6)
                return c2
            lax.fori_loop(0, bcol // 16, shf, 0)

            pltpu.async_copy(tab2_hbm.at[grp_v], gath_v, sem).wait()

            def dloop(d, c3):
                pv = plsc.load_gather(
                    pos_v, [lax.broadcast(s, (16,)), lax.broadcast(d, (16,))])
                for cc in range(bcol // 16):
                    kvec = lax.iota(jnp.int32, 16) + cc * 16
                    ovec = off_v[pl.ds(cc * 16, 16)] + d
                    vals = plsc.load_gather(gath_v, [kvec, ovec])
                    outt_v[d, pl.ds(cc * 16, 16)] = vals + pv
                return c3
            lax.fori_loop(0, D_MODEL, dloop, 0)

            pltpu.sync_copy(outt_v, out_hbm.at[s, :, pl.ds(b0, bcol)])
            return carry
        lax.fori_loop(0, SEQ_LEN, sloop, 0)

    return k(idxT, tab2, pos_enc)


def kernel(indices, table, pos_enc):
    idxT = indices.T.astype(jnp.int32)
    tab2 = table.reshape(VOCAB // 2, 2 * D_MODEL)
    out = _sc_call(idxT, tab2, pos_enc)
    return out.transpose(2, 0, 1)


# speedup vs baseline: 1.5660x; 1.5660x over previous
import functools

import jax
import jax.numpy as jnp
from jax import lax
from jax.experimental import pallas as pl
from jax.experimental.pallas import tpu as pltpu
from jax.experimental.pallas import tpu_sc as plsc

VOCAB = 1000000
SEQ_LEN = 200
D_MODEL = 64
BATCH = 4096


def _sc_call(idxT, tab2, pos_enc):
    info = plsc.get_sparse_core_info()
    nc, ns = info.num_cores, info.num_subcores
    nw = nc * ns
    bcol = BATCH // nw
    ncc = bcol // 16

    mesh = plsc.VectorSubcoreMesh(core_axis_name="c", subcore_axis_name="s")

    @functools.partial(
        pl.kernel,
        out_type=jax.ShapeDtypeStruct((SEQ_LEN, D_MODEL, BATCH), jnp.float32),
        mesh=mesh,
        scratch_types=[
            pltpu.VMEM((SEQ_LEN, bcol), jnp.int32),
            pltpu.VMEM((bcol,), jnp.int32),
            pltpu.VMEM((bcol,), jnp.int32),
            pltpu.VMEM((bcol, 2 * D_MODEL), jnp.float32),
            pltpu.VMEM((bcol, 2 * D_MODEL), jnp.float32),
            pltpu.VMEM((D_MODEL, bcol), jnp.float32),
            pltpu.VMEM((D_MODEL, bcol), jnp.float32),
            pltpu.VMEM((SEQ_LEN, D_MODEL), jnp.float32),
            pltpu.SemaphoreType.DMA,
            pltpu.SemaphoreType.DMA,
            pltpu.SemaphoreType.DMA,
            pltpu.SemaphoreType.DMA,
        ],
        compiler_params=pltpu.CompilerParams(
            use_tc_tiling_on_sc=True, needs_layout_passes=False),
    )
    def k(idxT_hbm, tab2_hbm, pos_hbm, out_hbm, idx_v, grp0, grp1,
          gath0, gath1, outt0, outt1, pos_v, gsem0, gsem1, osem0, osem1):
        wid = lax.axis_index("s") * nc + lax.axis_index("c")
        b0 = pl.multiple_of(wid * bcol, bcol)
        grp = (grp0, grp1)
        gath = (gath0, gath1)
        outt = (outt0, outt1)
        gsem = (gsem0, gsem1)
        osem = (osem0, osem1)

        pltpu.sync_copy(idxT_hbm.at[:, pl.ds(b0, bcol)], idx_v)
        pltpu.sync_copy(pos_hbm, pos_v)

        def issue_gather(s, p):
            for cc in range(ncc):
                v = idx_v[s, pl.ds(cc * 16, 16)]
                grp[p][pl.ds(cc * 16, 16)] = lax.shift_right_logical(v, 1)
            pltpu.async_copy(tab2_hbm.at[grp[p]], gath[p], gsem[p])

        def wait_gather(p):
            pltpu.make_async_copy(tab2_hbm.at[grp[p]], gath[p], gsem[p]).wait()

        def issue_out(s, p):
            pltpu.async_copy(outt[p], out_hbm.at[s, :, pl.ds(b0, bcol)],
                             osem[p])

        def wait_out(s, p):
            pltpu.make_async_copy(outt[p],
                                  out_hbm.at[s, :, pl.ds(b0, bcol)],
                                  osem[p]).wait()

        def compute(s, p):
            offs, kvecs = [], []
            for cc in range(ncc):
                v = idx_v[s, pl.ds(cc * 16, 16)]
                offs.append(lax.shift_left(lax.bitwise_and(v, 1), 6))
                kvecs.append(lax.iota(jnp.int32, 16) + cc * 16)
            sbc = lax.broadcast(s, (16,))

            def dbody(d, carry):
                offs_c, kvecs_c = carry
                pv = plsc.load_gather(pos_v, [sbc, lax.broadcast(d, (16,))])
                for cc in range(ncc):
                    ovec = offs_c[cc] + d
                    vals = plsc.load_gather(gath[p], [kvecs_c[cc], ovec])
                    outt[p][d, pl.ds(cc * 16, 16)] = vals + pv
                return carry
            lax.fori_loop(0, D_MODEL, dbody, (tuple(offs), tuple(kvecs)))

        issue_gather(0, 0)
        issue_gather(1, 1)
        wait_gather(0)
        compute(0, 0)
        issue_gather(2, 0)
        issue_out(0, 0)
        wait_gather(1)
        compute(1, 1)
        issue_gather(3, 1)
        issue_out(1, 1)

        def sbody(sp, carry):
            s0 = sp * 2
            for p in range(2):
                s = s0 + p
                wait_gather(p)
                wait_out(s - 2, p)
                compute(s, p)
                issue_gather(s + 2, p)
                issue_out(s, p)
            return carry
        lax.fori_loop(1, SEQ_LEN // 2 - 1, sbody, 0)

        for p, s in ((0, SEQ_LEN - 2), (1, SEQ_LEN - 1)):
            wait_gather(p)
            wait_out(s - 2, p)
            compute(s, p)
            issue_out(s, p)
        wait_out(SEQ_LEN - 2, 0)
        wait_out(SEQ_LEN - 1, 1)

    return k(idxT, tab2, pos_enc)


def kernel(indices, table, pos_enc):
    idxT = indices.T.astype(jnp.int32)
    tab2 = table.reshape(VOCAB // 2, 2 * D_MODEL)
    out = _sc_call(idxT, tab2, pos_enc)
    return out.transpose(2, 0, 1)
